# Initial kernel scaffold; baseline (speedup 1.0000x reference)
import jax
import jax.numpy as jnp
from jax.experimental import pallas as pl


def kernel(node_embeddings, node_types, layers, operators, in_degs, out_degs, negs, W_proj, b_proj, type_emb, layer_emb, op_emb, in_emb, out_emb, neg_W, neg_b, ln_gamma, ln_beta):
    raise NotImplementedError("write your pallas kernel here")


# speedup vs baseline: 6.8024x; 6.8024x over previous
import functools

import jax
import jax.numpy as jnp
from jax.experimental import pallas as pl

EPS = 1e-12
BLOCK = 2000


def _fused_body(emb_ref, idx_ref, wp_ref, tab_ref, nw_ref, bp_ref, nb_ref,
                g_ref, b_ref, out_ref):
    x = jnp.dot(emb_ref[...], wp_ref[...], preferred_element_type=jnp.float32)
    x = x + bp_ref[...]

    idx = idx_ref[...]
    b = x.shape[0]
    iota = jax.lax.broadcasted_iota(jnp.int32, (b, 32), 1)
    oh = ((iota == idx[:, 0:1])
          | (iota == idx[:, 1:2] + 2)
          | (iota == idx[:, 2:3] + 10)
          | (iota == idx[:, 3:4] + 12)
          | (iota == idx[:, 4:5] + 20)).astype(jnp.float32)
    x = x + jnp.dot(oh, tab_ref[...], preferred_element_type=jnp.float32)

    neg = jnp.dot(x, nw_ref[...], preferred_element_type=jnp.float32) + nb_ref[...]
    x = jnp.where(idx[:, 5:6] == 1, neg, x)

    mu = jnp.mean(x, axis=-1, keepdims=True)
    xc = x - mu
    var = jnp.mean(xc * xc, axis=-1, keepdims=True)
    out_ref[...] = xc / jnp.sqrt(var + EPS) * g_ref[...] + b_ref[...]


@functools.partial(jax.jit, static_argnames=())
def kernel(node_embeddings, node_types, layers, operators, in_degs, out_degs,
           negs, W_proj, b_proj, type_emb, layer_emb, op_emb, in_emb, out_emb,
           neg_W, neg_b, ln_gamma, ln_beta):
    n, d = node_embeddings.shape
    h = W_proj.shape[1]
    grid = n // BLOCK

    idx = jnp.stack([node_types, layers, operators, in_degs, out_degs, negs,
                     jnp.zeros_like(negs), jnp.zeros_like(negs)], axis=1)
    tables = jnp.zeros((32, h), jnp.float32)
    tables = tables.at[0:2].set(type_emb).at[2:10].set(layer_emb)
    tables = tables.at[10:12].set(op_emb).at[12:20].set(in_emb)
    tables = tables.at[20:28].set(out_emb)

    row = lambda i: (i, 0)
    rep = lambda i: (0, 0)
    return pl.pallas_call(
        _fused_body,
        grid=(grid,),
        in_specs=[
            pl.BlockSpec((BLOCK, d), row),
            pl.BlockSpec((BLOCK, 8), row),
            pl.BlockSpec((d, h), rep),
            pl.BlockSpec((32, h), rep),
            pl.BlockSpec((h, h), rep),
            pl.BlockSpec((1, h), rep),
            pl.BlockSpec((1, h), rep),
            pl.BlockSpec((1, h), rep),
            pl.BlockSpec((1, h), rep),
        ],
        out_specs=pl.BlockSpec((BLOCK, h), row),
        out_shape=jax.ShapeDtypeStruct((n, h), jnp.float32),
    )(node_embeddings, idx, W_proj, tables, neg_W,
      b_proj.reshape(1, h), neg_b.reshape(1, h),
      ln_gamma.reshape(1, h), ln_beta.reshape(1, h))


# speedup vs baseline: 8.1802x; 1.2025x over previous
import functools

import jax
import jax.numpy as jnp
from jax.experimental import pallas as pl

EPS = 1e-12
BLOCK = 2000

_OFFS = (0, 2, 10, 12, 20, 28, 30, 30)


def _fused_body(emb_ref, idx_ref, w_ref, j_ref, g_ref, b_ref, out_ref):
    idx = idx_ref[...]
    bsz = idx.shape[0]
    p = jnp.left_shift(jnp.int32(1), idx)
    r4 = p[:, 0:4] | p[:, 4:8]
    r2 = r4[:, 0:2] | r4[:, 2:4]
    bits = r2[:, 0:1] | r2[:, 1:2]
    iota = jax.lax.broadcasted_iota(jnp.int32, (bsz, 32), 1)
    oh = (jnp.right_shift(bits, iota) & 1).astype(jnp.float32)

    cat = jnp.concatenate([emb_ref[...], oh], axis=1)
    both = jnp.dot(cat, w_ref[...], preferred_element_type=jnp.float32)
    x = jnp.where(idx[:, 5:6] == 29, both[:, 128:256], both[:, 0:128])

    mu = jnp.dot(x, j_ref[...], preferred_element_type=jnp.float32)
    s2 = jnp.dot(x * x, j_ref[...], preferred_element_type=jnp.float32)
    var = s2 - mu * mu
    rstd = jax.lax.rsqrt(var + EPS)
    rg = rstd * g_ref[...]
    out_ref[...] = (x - mu) * rg + b_ref[...]


@functools.partial(jax.jit, static_argnames=())
def kernel(node_embeddings, node_types, layers, operators, in_degs, out_degs,
           negs, W_proj, b_proj, type_emb, layer_emb, op_emb, in_emb, out_emb,
           neg_W, neg_b, ln_gamma, ln_beta):
    n, d = node_embeddings.shape
    h = W_proj.shape[1]
    grid = n // BLOCK

    idx = jnp.stack([node_types, layers + 2, operators + 10, in_degs + 12,
                     out_degs + 20, negs + 28,
                     jnp.full_like(negs, 30), jnp.full_like(negs, 30)], axis=1)

    tables = jnp.zeros((32, h), jnp.float32)
    tables = tables.at[0:2].set(type_emb).at[2:10].set(layer_emb)
    tables = tables.at[10:12].set(op_emb).at[12:20].set(in_emb)
    tables = tables.at[20:28].set(out_emb)
    tables = tables.at[28].set(b_proj).at[29].set(b_proj)
    w2 = W_proj @ neg_W
    t2 = tables @ neg_W
    t2 = t2.at[28].add(neg_b).at[29].add(neg_b)
    w_cat = jnp.concatenate(
        [jnp.concatenate([W_proj, w2], axis=1),
         jnp.concatenate([tables, t2], axis=1)], axis=0)

    jmat = jnp.full((h, h), 1.0 / h, jnp.float32)

    row = lambda i: (i, 0)
    rep = lambda i: (0, 0)
    return pl.pallas_call(
        _fused_body,
        grid=(grid,),
        in_specs=[
            pl.BlockSpec((BLOCK, d), row),
            pl.BlockSpec((BLOCK, 8), row),
            pl.BlockSpec((d + 32, 2 * h), rep),
            pl.BlockSpec((h, h), rep),
            pl.BlockSpec((1, h), rep),
            pl.BlockSpec((1, h), rep),
        ],
        out_specs=pl.BlockSpec((BLOCK, h), row),
        out_shape=jax.ShapeDtypeStruct((n, h), jnp.float32),
    )(node_embeddings, idx, w_cat, jmat,
      ln_gamma.reshape(1, h), ln_beta.reshape(1, h))


# speedup vs baseline: 10.3878x; 1.2699x over previous
import functools

import jax
import jax.numpy as jnp
from jax.experimental import pallas as pl

EPS = 1e-12
BLOCK = 2000


def _fused_body(emb_ref, idx_ref, w_ref, t_ref, j_ref, g_ref, b_ref, out_ref):
    idx = idx_ref[...]
    bsz = idx.shape[0]
    bits = idx[:, 0:1]
    iota = jax.lax.broadcasted_iota(jnp.int32, (bsz, 32), 1)
    oh = (jnp.right_shift(bits, iota) & 1).astype(jnp.float32)

    y0 = jnp.dot(emb_ref[...], w_ref[...], preferred_element_type=jnp.float32)
    y1 = jnp.dot(oh, t_ref[...], preferred_element_type=jnp.float32)
    both = y0 + y1
    x = jnp.where(idx[:, 1:2] == 1, both[:, 128:256], both[:, 0:128])

    mu = jnp.dot(x, j_ref[...], preferred_element_type=jnp.float32)
    s2 = jnp.dot(x * x, j_ref[...], preferred_element_type=jnp.float32)
    var = s2 - mu * mu
    rstd = jax.lax.rsqrt(var + EPS)
    rg = rstd * g_ref[...]
    out_ref[...] = (x - mu) * rg + b_ref[...]


@functools.partial(jax.jit, static_argnames=())
def kernel(node_embeddings, node_types, layers, operators, in_degs, out_degs,
           negs, W_proj, b_proj, type_emb, layer_emb, op_emb, in_emb, out_emb,
           neg_W, neg_b, ln_gamma, ln_beta):
    n, d = node_embeddings.shape
    h = W_proj.shape[1]
    grid = n // BLOCK

    one = jnp.int32(1)
    bits = ((one << node_types) | (one << (layers + 2)) | (one << (operators + 10))
            | (one << (in_degs + 12)) | (one << (out_degs + 20))
            | (one << (negs + 28)))
    idx = jnp.stack([bits, negs], axis=1)

    tables = jnp.zeros((32, h), jnp.float32)
    tables = tables.at[0:2].set(type_emb).at[2:10].set(layer_emb)
    tables = tables.at[10:12].set(op_emb).at[12:20].set(in_emb)
    tables = tables.at[20:28].set(out_emb)
    tables = tables.at[28].set(b_proj).at[29].set(b_proj)
    w2 = W_proj @ neg_W
    t2 = tables @ neg_W
    t2 = t2.at[28].add(neg_b).at[29].add(neg_b)
    w_cat = jnp.concatenate([W_proj, w2], axis=1)
    t_cat = jnp.concatenate([tables, t2], axis=1)

    jmat = jnp.full((h, h), 1.0 / h, jnp.float32)

    row = lambda i: (i, 0)
    rep = lambda i: (0, 0)
    return pl.pallas_call(
        _fused_body,
        grid=(grid,),
        in_specs=[
            pl.BlockSpec((BLOCK, d), row),
            pl.BlockSpec((BLOCK, 2), row),
            pl.BlockSpec((d, 2 * h), rep),
            pl.BlockSpec((32, 2 * h), rep),
            pl.BlockSpec((h, h), rep),
            pl.BlockSpec((1, h), rep),
            pl.BlockSpec((1, h), rep),
        ],
        out_specs=pl.BlockSpec((BLOCK, h), row),
        out_shape=jax.ShapeDtypeStruct((n, h), jnp.float32),
    )(node_embeddings, idx, w_cat, t_cat, jmat,
      ln_gamma.reshape(1, h), ln_beta.reshape(1, h))


# speedup vs baseline: 11.7779x; 1.1338x over previous
import functools

import jax
import jax.numpy as jnp
from jax.experimental import pallas as pl

EPS = 1e-12
BLOCK = 2000


def _fused_body(emb_ref, t_ref, l_ref, o_ref, i_ref, u_ref, n_ref,
                w_ref, tab_ref, j_ref, g_ref, b_ref, out_ref):
    one = jnp.int32(1)
    bits = ((one << t_ref[0]) | (one << (l_ref[0] + 2))
            | (one << (o_ref[0] + 10)) | (one << (i_ref[0] + 12))
            | (one << (u_ref[0] + 20)) | (one << (n_ref[0] + 28)))
    bsz = bits.shape[1]
    iota = jax.lax.broadcasted_iota(jnp.int32, (32, bsz), 0)
    oh_t = (jnp.right_shift(jnp.broadcast_to(bits, (32, bsz)), iota)
            & 1).astype(jnp.float32)

    y0 = jnp.dot(emb_ref[...], w_ref[...], preferred_element_type=jnp.float32)
    y1 = jax.lax.dot_general(oh_t, tab_ref[...], (((0,), (0,)), ((), ())),
                             preferred_element_type=jnp.float32)
    both = y0 + y1[:, 0:256]
    x = jnp.where(y1[:, 256:257] > 0.5, both[:, 128:256], both[:, 0:128])

    mu = jnp.dot(x, j_ref[...], preferred_element_type=jnp.float32)
    s2 = jnp.dot(x * x, j_ref[...], preferred_element_type=jnp.float32)
    var = s2 - mu * mu
    rstd = jax.lax.rsqrt(var + EPS)
    rg = rstd * g_ref[...]
    out_ref[...] = (x - mu) * rg + b_ref[...]


@functools.partial(jax.jit, static_argnames=())
def kernel(node_embeddings, node_types, layers, operators, in_degs, out_degs,
           negs, W_proj, b_proj, type_emb, layer_emb, op_emb, in_emb, out_emb,
           neg_W, neg_b, ln_gamma, ln_beta):
    n, d = node_embeddings.shape
    h = W_proj.shape[1]
    grid = n // BLOCK

    tables = jnp.zeros((32, h), jnp.float32)
    tables = tables.at[0:2].set(type_emb).at[2:10].set(layer_emb)
    tables = tables.at[10:12].set(op_emb).at[12:20].set(in_emb)
    tables = tables.at[20:28].set(out_emb)
    tables = tables.at[28].set(b_proj).at[29].set(b_proj)
    w2 = W_proj @ neg_W
    t2 = tables @ neg_W
    t2 = t2.at[28].add(neg_b).at[29].add(neg_b)
    w_cat = jnp.concatenate([W_proj, w2], axis=1)
    ind = jnp.zeros((32, h), jnp.float32).at[29, 0].set(1.0)
    t_cat = jnp.concatenate([tables, t2, ind], axis=1)

    jmat = jnp.full((h, h), 1.0 / h, jnp.float32)

    row = lambda i: (i, 0)
    vec = lambda i: (i, 0, 0)
    rep = lambda i: (0, 0)
    ints = [a.reshape(grid, 1, BLOCK)
            for a in (node_types, layers, operators, in_degs, out_degs, negs)]
    return pl.pallas_call(
        _fused_body,
        grid=(grid,),
        in_specs=[
            pl.BlockSpec((BLOCK, d), row),
            pl.BlockSpec((1, 1, BLOCK), vec),
            pl.BlockSpec((1, 1, BLOCK), vec),
            pl.BlockSpec((1, 1, BLOCK), vec),
            pl.BlockSpec((1, 1, BLOCK), vec),
            pl.BlockSpec((1, 1, BLOCK), vec),
            pl.BlockSpec((1, 1, BLOCK), vec),
            pl.BlockSpec((d, 2 * h), rep),
            pl.BlockSpec((32, 3 * h), rep),
            pl.BlockSpec((h, h), rep),
            pl.BlockSpec((1, h), rep),
            pl.BlockSpec((1, h), rep),
        ],
        out_specs=pl.BlockSpec((BLOCK, h), row),
        out_shape=jax.ShapeDtypeStruct((n, h), jnp.float32),
    )(node_embeddings, *ints, w_cat, t_cat, jmat,
      ln_gamma.reshape(1, h), ln_beta.reshape(1, h))


# speedup vs baseline: 13.0726x; 1.1099x over previous
import functools

import jax
import jax.numpy as jnp
from jax.experimental import pallas as pl
from jax.experimental.pallas import tpu as pltpu

EPS = 1e-12
BLOCK = 2000


def _fused_body(emb_ref, t_ref, l_ref, o_ref, i_ref, u_ref, n_ref,
                wp_ref, te_ref, le_ref, oe_ref, ie_ref, ue_ref,
                nw_ref, bp_ref, nb_ref, g_ref, b_ref,
                out_ref, w_ref, tab_ref, j_ref):
    @pl.when(pl.program_id(0) == 0)
    def _prologue():
        h = wp_ref.shape[1]
        zero2 = jnp.zeros((1, h), jnp.float32)
        tab = jnp.concatenate(
            [te_ref[...], le_ref[...], oe_ref[...], ie_ref[...], ue_ref[...],
             bp_ref[...], bp_ref[...], zero2, zero2], axis=0)
        t2 = jnp.dot(tab, nw_ref[...], preferred_element_type=jnp.float32)
        si = jax.lax.broadcasted_iota(jnp.int32, (32, h), 0)
        li = jax.lax.broadcasted_iota(jnp.int32, (32, h), 1)
        brow = ((si == 28) | (si == 29)).astype(jnp.float32)
        t2 = t2 + brow * nb_ref[...]
        ind = ((si == 29) & (li == 0)).astype(jnp.float32)
        tab_ref[...] = jnp.concatenate([tab, t2, ind], axis=1)
        w2 = jnp.dot(wp_ref[...], nw_ref[...],
                     preferred_element_type=jnp.float32)
        w_ref[...] = jnp.concatenate([wp_ref[...], w2], axis=1)
        j_ref[...] = jnp.full(j_ref.shape, 1.0 / h, jnp.float32)

    one = jnp.int32(1)
    bits = ((one << t_ref[0]) | (one << (l_ref[0] + 2))
            | (one << (o_ref[0] + 10)) | (one << (i_ref[0] + 12))
            | (one << (u_ref[0] + 20)) | (one << (n_ref[0] + 28)))
    bsz = bits.shape[1]
    iota = jax.lax.broadcasted_iota(jnp.int32, (32, bsz), 0)
    oh_t = (jnp.right_shift(jnp.broadcast_to(bits, (32, bsz)), iota)
            & 1).astype(jnp.float32)

    y0 = jnp.dot(emb_ref[...], w_ref[...], preferred_element_type=jnp.float32)
    y1 = jax.lax.dot_general(oh_t, tab_ref[...], (((0,), (0,)), ((), ())),
                             preferred_element_type=jnp.float32)
    both = y0 + y1[:, 0:256]
    x = jnp.where(y1[:, 256:257] > 0.5, both[:, 128:256], both[:, 0:128])

    mu = jnp.dot(x, j_ref[...], preferred_element_type=jnp.float32)
    s2 = jnp.dot(x * x, j_ref[...], preferred_element_type=jnp.float32)
    var = s2 - mu * mu
    rstd = jax.lax.rsqrt(var + EPS)
    rg = rstd * g_ref[...]
    out_ref[...] = (x - mu) * rg + b_ref[...]


@functools.partial(jax.jit, static_argnames=())
def kernel(node_embeddings, node_types, layers, operators, in_degs, out_degs,
           negs, W_proj, b_proj, type_emb, layer_emb, op_emb, in_emb, out_emb,
           neg_W, neg_b, ln_gamma, ln_beta):
    n, d = node_embeddings.shape
    h = W_proj.shape[1]
    grid = n // BLOCK

    row = lambda i: (i, 0)
    vec = lambda i: (i, 0, 0)
    rep = lambda i: (0, 0)
    ints = [a.reshape(grid, 1, BLOCK)
            for a in (node_types, layers, operators, in_degs, out_degs, negs)]
    return pl.pallas_call(
        _fused_body,
        grid=(grid,),
        in_specs=[
            pl.BlockSpec((BLOCK, d), row),
            pl.BlockSpec((1, 1, BLOCK), vec),
            pl.BlockSpec((1, 1, BLOCK), vec),
            pl.BlockSpec((1, 1, BLOCK), vec),
            pl.BlockSpec((1, 1, BLOCK), vec),
            pl.BlockSpec((1, 1, BLOCK), vec),
            pl.BlockSpec((1, 1, BLOCK), vec),
            pl.BlockSpec((d, h), rep),
            pl.BlockSpec((2, h), rep),
            pl.BlockSpec((8, h), rep),
            pl.BlockSpec((2, h), rep),
            pl.BlockSpec((8, h), rep),
            pl.BlockSpec((8, h), rep),
            pl.BlockSpec((h, h), rep),
            pl.BlockSpec((1, h), rep),
            pl.BlockSpec((1, h), rep),
            pl.BlockSpec((1, h), rep),
            pl.BlockSpec((1, h), rep),
        ],
        out_specs=pl.BlockSpec((BLOCK, h), row),
        out_shape=jax.ShapeDtypeStruct((n, h), jnp.float32),
        scratch_shapes=[
            pltpu.VMEM((d, 2 * h), jnp.float32),
            pltpu.VMEM((32, 3 * h), jnp.float32),
            pltpu.VMEM((h, h), jnp.float32),
        ],
    )(node_embeddings, *ints, W_proj, type_emb, layer_emb, op_emb,
      in_emb, out_emb, neg_W, b_proj.reshape(1, h), neg_b.reshape(1, h),
      ln_gamma.reshape(1, h), ln_beta.reshape(1, h))


# speedup vs baseline: 14.9376x; 1.1427x over previous
import functools

import jax
import jax.numpy as jnp
from jax.experimental import pallas as pl
from jax.experimental.pallas import tpu as pltpu

EPS = 1e-12
BLOCK = 2048


def _fused_body(emb_ref, t_ref, l_ref, o_ref, i_ref, u_ref, n_ref,
                wp_ref, te_ref, le_ref, oe_ref, ie_ref, ue_ref,
                nw_ref, bp_ref, nb_ref, g_ref, b_ref,
                out_ref, w_ref, tab_ref, j_ref):
    @pl.when(pl.program_id(0) == 0)
    def _prologue():
        h = wp_ref.shape[1]
        zero2 = jnp.zeros((1, h), jnp.float32)
        tab = jnp.concatenate(
            [te_ref[...], le_ref[...], oe_ref[...], ie_ref[...], ue_ref[...],
             bp_ref[...], bp_ref[...], zero2, zero2], axis=0)
        t2 = jnp.dot(tab, nw_ref[...], preferred_element_type=jnp.float32)
        si = jax.lax.broadcasted_iota(jnp.int32, (32, h), 0)
        li = jax.lax.broadcasted_iota(jnp.int32, (32, h), 1)
        brow = ((si == 28) | (si == 29)).astype(jnp.float32)
        t2 = t2 + brow * nb_ref[...]
        ind = ((si == 29) & (li == 0)).astype(jnp.float32)
        tab_ref[...] = jnp.concatenate([tab, t2, ind], axis=1)
        w2 = jnp.dot(wp_ref[...], nw_ref[...],
                     preferred_element_type=jnp.float32)
        w_ref[...] = jnp.concatenate([wp_ref[...], w2], axis=1)
        j_ref[...] = jnp.full(j_ref.shape, 1.0 / h, jnp.float32)

    one = jnp.int32(1)
    bits = ((one << t_ref[...]) | (one << (l_ref[...] + 2))
            | (one << (o_ref[...] + 10)) | (one << (i_ref[...] + 12))
            | (one << (u_ref[...] + 20)) | (one << (n_ref[...] + 28)))
    bsz = bits.shape[0]
    iota = jax.lax.broadcasted_iota(jnp.int32, (32, bsz), 0)
    oh_t = (jnp.right_shift(jnp.broadcast_to(bits, (32, bsz)), iota)
            & 1).astype(jnp.float32)

    y0 = jnp.dot(emb_ref[...], w_ref[...], preferred_element_type=jnp.float32)
    y1 = jax.lax.dot_general(oh_t, tab_ref[...], (((0,), (0,)), ((), ())),
                             preferred_element_type=jnp.float32)
    both = y0 + y1[:, 0:256]
    x = jnp.where(y1[:, 256:257] > 0.5, both[:, 128:256], both[:, 0:128])

    mu = jnp.dot(x, j_ref[...], preferred_element_type=jnp.float32)
    s2 = jnp.dot(x * x, j_ref[...], preferred_element_type=jnp.float32)
    var = s2 - mu * mu
    rstd = jax.lax.rsqrt(var + EPS)
    rg = rstd * g_ref[...]
    out_ref[...] = (x - mu) * rg + b_ref[...]


@functools.partial(jax.jit, static_argnames=())
def kernel(node_embeddings, node_types, layers, operators, in_degs, out_degs,
           negs, W_proj, b_proj, type_emb, layer_emb, op_emb, in_emb, out_emb,
           neg_W, neg_b, ln_gamma, ln_beta):
    n, d = node_embeddings.shape
    h = W_proj.shape[1]
    grid = pl.cdiv(n, BLOCK)

    row = lambda i: (i, 0)
    vec = lambda i: (i,)
    rep = lambda i: (0, 0)
    ints = [node_types, layers, operators, in_degs, out_degs, negs]
    return pl.pallas_call(
        _fused_body,
        grid=(grid,),
        in_specs=[
            pl.BlockSpec((BLOCK, d), row),
            pl.BlockSpec((BLOCK,), vec),
            pl.BlockSpec((BLOCK,), vec),
            pl.BlockSpec((BLOCK,), vec),
            pl.BlockSpec((BLOCK,), vec),
            pl.BlockSpec((BLOCK,), vec),
            pl.BlockSpec((BLOCK,), vec),
            pl.BlockSpec((d, h), rep),
            pl.BlockSpec((2, h), rep),
            pl.BlockSpec((8, h), rep),
            pl.BlockSpec((2, h), rep),
            pl.BlockSpec((8, h), rep),
            pl.BlockSpec((8, h), rep),
            pl.BlockSpec((h, h), rep),
            pl.BlockSpec((1, h), rep),
            pl.BlockSpec((1, h), rep),
            pl.BlockSpec((1, h), rep),
            pl.BlockSpec((1, h), rep),
        ],
        out_specs=pl.BlockSpec((BLOCK, h), row),
        out_shape=jax.ShapeDtypeStruct((n, h), jnp.float32),
        scratch_shapes=[
            pltpu.VMEM((d, 2 * h), jnp.float32),
            pltpu.VMEM((32, 3 * h), jnp.float32),
            pltpu.VMEM((h, h), jnp.float32),
        ],
    )(node_embeddings, *ints, W_proj, type_emb, layer_emb, op_emb,
      in_emb, out_emb, neg_W, b_proj.reshape(1, h), neg_b.reshape(1, h),
      ln_gamma.reshape(1, h), ln_beta.reshape(1, h))
